# baseline (speedup 1.0000x reference)
import functools

import jax
import jax.numpy as jnp
from jax.experimental import pallas as pl
from jax.experimental.pallas import tpu as pltpu

_HD = 128
_K_LAT = 32
_KNOTS = 7
_EPS = 1e-4
_PACK = 4
_LANES = _PACK * _HD
_ROWS = 32768


def _fused_kernel(xk_ref, delta_ref, scale_raw_ref, shift_ref, x_mu_ref,
                  x_std_ref, mu_ref, w_ref, v_ref, o_ref):
    xk = xk_ref[...]
    seg_dx = xk[1:, :] - xk[:-1, :]
    slopes = jax.nn.softplus(delta_ref[...]) + _EPS
    avg = (jnp.sum(slopes * seg_dx, axis=0, keepdims=True)
           / (jnp.sum(seg_dx, axis=0, keepdims=True) + 1e-8))
    avg = jnp.maximum(avg, 1e-6)
    slopes = slopes / avg

    scale = jax.nn.softplus(scale_raw_ref[...]) + 1e-3
    x_std = x_std_ref[...]
    inv_std = 1.0 / x_std
    a = slopes * (inv_std * scale)
    t = xk * x_std + x_mu_ref[...]
    c = jnp.concatenate([a[0:1, :], a[1:, :] - a[:-1, :]], axis=0)
    const = shift_ref[...] - mu_ref[...] - jnp.sum(c * t[:-1, :], axis=0,
                                                   keepdims=True)

    v = v_ref[...]
    y = c[0:1, :] * v
    for j in range(1, _KNOTS - 1):
        y = y + c[j:j + 1, :] * jnp.maximum(v, t[j:j + 1, :])

    w = w_ref[...]
    bias = jnp.dot(const, w, preferred_element_type=jnp.float32)
    o_ref[...] = jnp.dot(y, w, preferred_element_type=jnp.float32) + bias


def kernel(V, xk, delta_raw, scale_raw, shift, x_mu, x_std, mu, Vk):
    lead = V.shape[:-1]
    n = V.size // _HD
    Vp = V.reshape(n // _PACK, _LANES)
    r4 = _ROWS // _PACK
    grid = (n // _ROWS,)

    tile = lambda p: jnp.tile(p.reshape(1, _HD), (1, _PACK))
    w = jnp.kron(jnp.eye(_PACK, dtype=jnp.float32), Vk)

    full = lambda shape: pl.BlockSpec(shape, lambda i: (0,) * len(shape))
    out = pl.pallas_call(
        _fused_kernel,
        grid=grid,
        in_specs=[
            full((_KNOTS, _LANES)),
            full((_KNOTS - 1, _LANES)),
            full((1, _LANES)),
            full((1, _LANES)),
            full((1, _LANES)),
            full((1, _LANES)),
            full((1, _LANES)),
            full((_LANES, _HD)),
            pl.BlockSpec((r4, _LANES), lambda i: (i, 0)),
        ],
        out_specs=pl.BlockSpec((r4, _HD), lambda i: (i, 0)),
        out_shape=jax.ShapeDtypeStruct((n // _PACK, _HD), jnp.float32),
        compiler_params=pltpu.CompilerParams(
            dimension_semantics=("parallel",)),
    )(jnp.tile(xk.T, (1, _PACK)), jnp.tile(delta_raw.T, (1, _PACK)),
      tile(scale_raw), tile(shift), tile(x_mu), tile(x_std), tile(mu),
      w, Vp)
    return out.reshape(lead + (_K_LAT,))


# speedup vs baseline: 2.0068x; 2.0068x over previous
import jax
import jax.numpy as jnp
from jax.experimental import pallas as pl
from jax.experimental.pallas import tpu as pltpu

_HD = 128
_K_LAT = 32
_KNOTS = 7
_EPS = 1e-4
_ROWS = 16384
_N_MXU = 2


def _hinge_coeffs(xk, delta_raw, scale_raw, x_std, x_mu, axis):
    seg_dx = jnp.diff(xk, axis=axis)
    slopes = jax.nn.softplus(delta_raw) + _EPS
    avg = (jnp.sum(slopes * seg_dx, axis=axis, keepdims=True)
           / (jnp.sum(seg_dx, axis=axis, keepdims=True) + 1e-8))
    slopes = slopes / jnp.maximum(avg, 1e-6)
    scale = jax.nn.softplus(scale_raw) + 1e-3
    a = slopes * (scale / x_std)
    t = xk * x_std + x_mu
    a0 = a[0:1, :] if axis == 0 else a[:, 0:1]
    c = jnp.concatenate([a0, jnp.diff(a, axis=axis)], axis=axis)
    return c, t


def _fused_kernel(xk_ref, delta_ref, scale_raw_ref, shift_ref, x_mu_ref,
                  x_std_ref, mu_ref, vk_ref, xk_c_ref, delta_c_ref,
                  scale_raw_c_ref, x_mu_c_ref, x_std_c_ref, v_ref, o_ref):
    c, t = _hinge_coeffs(xk_ref[...], delta_ref[...], scale_raw_ref[...],
                         x_std_ref[...], x_mu_ref[...], axis=0)
    const = shift_ref[...] - mu_ref[...] - jnp.sum(
        c * t[:-1, :], axis=0, keepdims=True)
    c_col, _ = _hinge_coeffs(xk_c_ref[...], delta_c_ref[...],
                             scale_raw_c_ref[...], x_std_c_ref[...],
                             x_mu_c_ref[...], axis=1)

    v = v_ref[...]
    n_vpu = _KNOTS - 1 - _N_MXU
    y = c[0:1, :] * v
    for j in range(1, n_vpu):
        y = y + c[j:j + 1, :] * jnp.maximum(v, t[j:j + 1, :])

    vk = vk_ref[...]
    bias = jnp.dot(const, vk, preferred_element_type=jnp.float32)
    acc = jnp.dot(y, vk, preferred_element_type=jnp.float32) + bias
    for j in range(n_vpu, _KNOTS - 1):
        h = jnp.maximum(v, t[j:j + 1, :])
        wj = c_col[:, j:j + 1] * vk
        acc = acc + jnp.dot(h, wj, preferred_element_type=jnp.float32)
    o_ref[...] = acc


def kernel(V, xk, delta_raw, scale_raw, shift, x_mu, x_std, mu, Vk):
    lead = V.shape[:-1]
    n = V.size // _HD
    V2 = V.reshape(n, _HD)
    grid = (n // _ROWS,)

    full = lambda shape: pl.BlockSpec(shape, lambda i: (0,) * len(shape))
    out = pl.pallas_call(
        _fused_kernel,
        grid=grid,
        in_specs=[
            full((_KNOTS, _HD)),
            full((_KNOTS - 1, _HD)),
            full((1, _HD)),
            full((1, _HD)),
            full((1, _HD)),
            full((1, _HD)),
            full((1, _HD)),
            full((_HD, _K_LAT)),
            full((_HD, _KNOTS)),
            full((_HD, _KNOTS - 1)),
            full((_HD, 1)),
            full((_HD, 1)),
            full((_HD, 1)),
            pl.BlockSpec((_ROWS, _HD), lambda i: (i, 0)),
        ],
        out_specs=pl.BlockSpec((_ROWS, _K_LAT), lambda i: (i, 0)),
        out_shape=jax.ShapeDtypeStruct((n, _K_LAT), jnp.float32),
        compiler_params=pltpu.CompilerParams(
            dimension_semantics=("parallel",)),
    )(xk.T, delta_raw.T, scale_raw.reshape(1, _HD), shift.reshape(1, _HD),
      x_mu, x_std, mu, Vk,
      xk, delta_raw, scale_raw.reshape(_HD, 1), x_mu.reshape(_HD, 1),
      x_std.reshape(_HD, 1), V2)
    return out.reshape(lead + (_K_LAT,))


# speedup vs baseline: 2.1259x; 1.0593x over previous
import jax
import jax.numpy as jnp
from jax.experimental import pallas as pl
from jax.experimental.pallas import tpu as pltpu

_HD = 128
_K_LAT = 32
_KNOTS = 7
_EPS = 1e-4
_ROWS = 16384
_HALF = _ROWS // 2


def _fused_kernel(xk_ref, delta_ref, scale_raw_ref, shift_ref, x_mu_ref,
                  x_std_ref, mu_ref, vk_ref, va_ref, vb_ref, o_ref):
    xk = xk_ref[...]
    seg_dx = xk[1:, :] - xk[:-1, :]
    slopes = jax.nn.softplus(delta_ref[...]) + _EPS
    avg = (jnp.sum(slopes * seg_dx, axis=0, keepdims=True)
           / (jnp.sum(seg_dx, axis=0, keepdims=True) + 1e-8))
    avg = jnp.maximum(avg, 1e-6)
    slopes = slopes / avg

    scale = jax.nn.softplus(scale_raw_ref[...]) + 1e-3
    x_std = x_std_ref[...]
    inv_std = 1.0 / x_std
    a = slopes * (inv_std * scale)
    t = xk * x_std + x_mu_ref[...]
    c = jnp.concatenate([a[0:1, :], a[1:, :] - a[:-1, :]], axis=0)
    const = shift_ref[...] - mu_ref[...] - jnp.sum(c * t[:-1, :], axis=0,
                                                   keepdims=True)

    vk = vk_ref[...]
    bias = jnp.dot(const, vk, preferred_element_type=jnp.float32)

    for v_ref_h, sl in ((va_ref, slice(0, _HALF)),
                        (vb_ref, slice(_HALF, _ROWS))):
        v = v_ref_h[...]
        y = c[0:1, :] * v
        for j in range(1, _KNOTS - 1):
            y = y + c[j:j + 1, :] * jnp.maximum(v, t[j:j + 1, :])
        o_ref[sl, :] = jnp.dot(y, vk, preferred_element_type=jnp.float32) + bias


def kernel(V, xk, delta_raw, scale_raw, shift, x_mu, x_std, mu, Vk):
    lead = V.shape[:-1]
    n = V.size // _HD
    V2 = V.reshape(n, _HD)
    grid = (n // _ROWS,)

    full = lambda shape: pl.BlockSpec(shape, lambda i: (0,) * len(shape))
    out = pl.pallas_call(
        _fused_kernel,
        grid=grid,
        in_specs=[
            full((_KNOTS, _HD)),
            full((_KNOTS - 1, _HD)),
            full((1, _HD)),
            full((1, _HD)),
            full((1, _HD)),
            full((1, _HD)),
            full((1, _HD)),
            full((_HD, _K_LAT)),
            pl.BlockSpec((_HALF, _HD), lambda i: (2 * i, 0)),
            pl.BlockSpec((_HALF, _HD), lambda i: (2 * i + 1, 0)),
        ],
        out_specs=pl.BlockSpec((_ROWS, _K_LAT), lambda i: (i, 0)),
        out_shape=jax.ShapeDtypeStruct((n, _K_LAT), jnp.float32),
        compiler_params=pltpu.CompilerParams(
            dimension_semantics=("parallel",)),
    )(xk.T, delta_raw.T, scale_raw.reshape(1, _HD), shift.reshape(1, _HD),
      x_mu, x_std, mu, Vk, V2, V2)
    return out.reshape(lead + (_K_LAT,))
